# baseline (speedup 1.0000x reference)
import functools

import jax
import jax.numpy as jnp
from jax import lax
from jax.experimental import pallas as pl
from jax.experimental.pallas import tpu as pltpu
from jax.experimental.pallas import tpu_sc as plsc

B = 16384
K = 32
NC = 2
NS = 16
L = 16
NW = NC * NS
BPW = B // NW
CH = 128
NCH = BPW // CH
RPV = 4
NV = 250000


def _body(uid_hbm, iid_hbm, pi4_hbm, eta4_hbm, out_hbm,
          uid_v, iid_v, out_v, stash_v,
          rbu_a, rbu_b, rbt_a, rbt_b,
          pi_a, pi_b, eta_a, eta_b, sem_a, sem_b):
    wid = lax.axis_index("s") * NC + lax.axis_index("c")

    pltpu.sync_copy(uid_hbm.at[wid], uid_v)
    pltpu.sync_copy(iid_hbm.at[wid], iid_v)

    last_lane = lax.iota(jnp.int32, L) * L + (L - 1)

    def fire(c, rbu, rbt, pi_buf, eta_buf, sem):
        for g in range(CH // L):
            s = pl.ds(g * L, L)
            rbu[s] = uid_v[pl.ds(c * CH + g * L, L)] >> 2
            rbt[s] = iid_v[pl.ds(c * CH + g * L, L)] >> 2
        pltpu.async_copy(pi4_hbm.at[rbu], pi_buf, sem)
        pltpu.async_copy(eta4_hbm.at[rbt], eta_buf, sem)

    def drain(pi_buf, eta_buf, sem):
        pltpu.make_async_copy(pi4_hbm.at[pl.ds(0, CH)], pi_buf, sem).wait()
        pltpu.make_async_copy(eta4_hbm.at[pl.ds(0, CH)], eta_buf, sem).wait()

    def compute(c, pi_buf, eta_buf):
        def group(g, carry):
            base = g * L
            uvec = uid_v[pl.ds(c * CH + base, L)]
            tvec = iid_v[pl.ds(c * CH + base, L)]
            for j in range(L):
                qu = pl.multiple_of((uvec[j] & 3) * K, K)
                qt = pl.multiple_of((tvec[j] & 3) * K, K)
                r = base + j
                v = (pi_buf[r, pl.ds(qu, L)] * eta_buf[r, pl.ds(qt, L)]
                     + pi_buf[r, pl.ds(qu + L, L)]
                     * eta_buf[r, pl.ds(qt + L, L)])
                stash_v[pl.ds(j * L, L)] = plsc.cumsum(v)
            sums = plsc.load_gather(stash_v, [last_lane])
            out_v[pl.ds(c * CH + base, L)] = 1.0 - jnp.exp(-sums)
            return carry

        lax.fori_loop(0, CH // L, group, 0)

    bufs = [(rbu_a, rbt_a, pi_a, eta_a, sem_a),
            (rbu_b, rbt_b, pi_b, eta_b, sem_b)]

    fire(0, *bufs[0])
    for c in range(NCH):
        if c + 1 < NCH:
            fire(c + 1, *bufs[(c + 1) % 2])
        rbu, rbt, pi_buf, eta_buf, sem = bufs[c % 2]
        drain(pi_buf, eta_buf, sem)
        compute(c, pi_buf, eta_buf)

    pltpu.sync_copy(out_v, out_hbm.at[pl.ds(wid * BPW, BPW)])


_pf = functools.partial(
    pl.kernel,
    mesh=plsc.VectorSubcoreMesh(core_axis_name="c", subcore_axis_name="s"),
    out_type=jax.ShapeDtypeStruct((B,), jnp.float32),
    compiler_params=pltpu.CompilerParams(needs_layout_passes=False),
    scratch_types=[
        pltpu.VMEM((BPW,), jnp.int32),
        pltpu.VMEM((BPW,), jnp.int32),
        pltpu.VMEM((BPW,), jnp.float32),
        pltpu.VMEM((L * L,), jnp.float32),
        pltpu.VMEM((CH,), jnp.int32),
        pltpu.VMEM((CH,), jnp.int32),
        pltpu.VMEM((CH,), jnp.int32),
        pltpu.VMEM((CH,), jnp.int32),
        pltpu.VMEM((CH, 4 * K), jnp.float32),
        pltpu.VMEM((CH, 4 * K), jnp.float32),
        pltpu.VMEM((CH, 4 * K), jnp.float32),
        pltpu.VMEM((CH, 4 * K), jnp.float32),
        pltpu.SemaphoreType.DMA,
        pltpu.SemaphoreType.DMA,
    ],
)(_body)


def kernel(user_ids, item_ids, pi, eta):
    uid = user_ids.astype(jnp.int32).reshape(NW, BPW)
    iid = item_ids.astype(jnp.int32).reshape(NW, BPW)
    pi4 = pi.reshape(NV, 4 * K)
    eta4 = eta.reshape(NV, 4 * K)
    return _pf(uid, iid, pi4, eta4)


# speedup vs baseline: 1.2689x; 1.2689x over previous
import functools

import jax
import jax.numpy as jnp
from jax import lax
from jax.experimental import pallas as pl
from jax.experimental.pallas import tpu as pltpu

B = 16384
K = 32


def _tc_body(uid_smem, iid_smem, pi_hbm, eta_hbm, out_vmem,
             pi_rows, eta_rows, sem_pi, sem_eta):
    def fire(i, carry):
        u = uid_smem[i]
        t = iid_smem[i]
        pltpu.make_async_copy(
            pi_hbm.at[pl.ds(u, 1)], pi_rows.at[pl.ds(i, 1)], sem_pi
        ).start()
        pltpu.make_async_copy(
            eta_hbm.at[pl.ds(t, 1)], eta_rows.at[pl.ds(i, 1)], sem_eta
        ).start()
        return carry

    lax.fori_loop(0, B, fire, 0, unroll=8)

    pltpu.make_async_copy(pi_hbm.at[pl.ds(0, B)], pi_rows, sem_pi).wait()
    pltpu.make_async_copy(eta_hbm.at[pl.ds(0, B)], eta_rows, sem_eta).wait()

    lam = jnp.sum(pi_rows[...] * eta_rows[...], axis=1)
    out_vmem[...] = 1.0 - jnp.exp(-lam)


@functools.partial(jax.jit, static_argnames=())
def _tc_call(uid, iid, pi, eta):
    grid_spec = pltpu.PrefetchScalarGridSpec(
        num_scalar_prefetch=2,
        grid=(1,),
        in_specs=[
            pl.BlockSpec(memory_space=pl.ANY),
            pl.BlockSpec(memory_space=pl.ANY),
        ],
        out_specs=pl.BlockSpec(memory_space=pltpu.VMEM),
        scratch_shapes=[
            pltpu.VMEM((B, K), jnp.float32),
            pltpu.VMEM((B, K), jnp.float32),
            pltpu.SemaphoreType.DMA,
            pltpu.SemaphoreType.DMA,
        ],
    )
    return pl.pallas_call(
        _tc_body,
        grid_spec=grid_spec,
        out_shape=jax.ShapeDtypeStruct((B,), jnp.float32),
    )(uid, iid, pi, eta)


def kernel(user_ids, item_ids, pi, eta):
    return _tc_call(user_ids.astype(jnp.int32), item_ids.astype(jnp.int32),
                    pi, eta)
